# Initial kernel scaffold; baseline (speedup 1.0000x reference)
import jax
import jax.numpy as jnp
from jax.experimental import pallas as pl


def kernel(x, w_pw1, g_pw1, b_pw1, w_dw, g_dw, b_dw, w_pw2, g_pw2, b_pw2):
    raise NotImplementedError("write your pallas kernel here")


# speedup vs baseline: 3.3604x; 3.3604x over previous
import functools

import jax
import jax.numpy as jnp
from jax import lax
from jax.experimental import pallas as pl
from jax.experimental.pallas import tpu as pltpu

EPS = 1e-5
BM = 1024


def _colsums(ps_ref, z):
    ones = jnp.ones((1, z.shape[0]), jnp.float32)
    ps_ref[0:1, :] = jnp.dot(ones, z, preferred_element_type=jnp.float32)
    ps_ref[1:2, :] = jnp.dot(ones, z * z, preferred_element_type=jnp.float32)


def _pw1_kernel(x_ref, w_ref, z_ref, ps_ref):
    z = jnp.dot(x_ref[...], w_ref[...], preferred_element_type=jnp.float32)
    z_ref[...] = z.astype(jnp.bfloat16)
    _colsums(ps_ref, z)


def _dw_kernel(z_ref, ss_ref, w_ref, o_ref, ps_ref):
    h, w, cm = z_ref.shape
    hh = h // 2
    s = ss_ref[0:1, :]
    b = ss_ref[1:2, :]
    wt = w_ref[...]
    psum, psq = None, None
    ones = jnp.ones((1, hh * w), jnp.float32)
    for half in range(2):
        r0 = half * hh
        lo = max(r0 - 1, 0)
        hi = min(r0 + hh + 1, h)
        zs = z_ref[lo:hi]
        y = jnp.maximum(zs.astype(jnp.float32) * s + b, 0.0)
        yp = jnp.pad(y.astype(jnp.bfloat16),
                     ((lo - (r0 - 1), (r0 + hh + 1) - hi), (1, 1), (0, 0)))
        acc = jnp.zeros((hh, w, cm), jnp.float32)
        for kh in range(3):
            for kw in range(3):
                tap = yp[kh:kh + hh, kw:kw + w, :].astype(jnp.float32)
                acc = acc + tap * wt[kh * 3 + kw:kh * 3 + kw + 1, :]
        o_ref[r0:r0 + hh] = acc.astype(jnp.bfloat16)
        a2 = acc.reshape(hh * w, cm)
        ssum = jnp.dot(ones, a2, preferred_element_type=jnp.float32)
        ssq = jnp.dot(ones, a2 * a2, preferred_element_type=jnp.float32)
        psum = ssum if psum is None else psum + ssum
        psq = ssq if psq is None else psq + ssq
    ps_ref[0:1, :] = psum
    ps_ref[1:2, :] = psq


def _pw2_kernel(z_ref, ss_ref, w_ref, z3_ref, ps_ref):
    y = jnp.maximum(z_ref[...].astype(jnp.float32) * ss_ref[0:1, :]
                    + ss_ref[1:2, :], 0.0)
    z3 = jnp.dot(y.astype(jnp.bfloat16), w_ref[...],
                 preferred_element_type=jnp.float32)
    z3_ref[...] = z3
    _colsums(ps_ref, z3)


def _bn_res_kernel(z_ref, ss_ref, r_ref, o_ref):
    o_ref[...] = (jnp.maximum(z_ref[...] * ss_ref[0:1, :] + ss_ref[1:2, :],
                              0.0) + r_ref[...])


def _finalize(ps, g, b, count):
    s = ps.sum(axis=0)
    mu = s[0:1, :] / count
    var = s[1:2, :] / count - mu * mu
    scale = g * lax.rsqrt(var + EPS)
    shift = b - mu * scale
    return jnp.concatenate([scale, shift], axis=0)


def _parallel(n):
    return pltpu.CompilerParams(dimension_semantics=("parallel",) * n)


def kernel(x, w_pw1, g_pw1, b_pw1, w_dw, g_dw, b_dw, w_pw2, g_pw2, b_pw2):
    n, c, h, w = x.shape
    cm = w_pw1.shape[1]
    co = w_pw2.shape[1]
    m = n * h * w
    bm = BM if m % BM == 0 else w * (h // 2)
    nt = m // bm

    x_nhwc = jnp.transpose(x, (0, 2, 3, 1))
    x2d = x_nhwc.reshape(m, c)
    xb = x2d.astype(jnp.bfloat16)
    w1b = w_pw1.astype(jnp.bfloat16)
    w2b = w_pw2.astype(jnp.bfloat16)
    w9 = w_dw.reshape(9, cm)

    z1, ps1 = pl.pallas_call(
        _pw1_kernel, grid=(nt,),
        in_specs=[pl.BlockSpec((bm, c), lambda i: (i, 0)),
                  pl.BlockSpec((c, cm), lambda i: (0, 0))],
        out_specs=[pl.BlockSpec((bm, cm), lambda i: (i, 0)),
                   pl.BlockSpec((None, 2, cm), lambda i: (i, 0, 0))],
        out_shape=[jax.ShapeDtypeStruct((m, cm), jnp.bfloat16),
                   jax.ShapeDtypeStruct((nt, 2, cm), jnp.float32)],
        compiler_params=_parallel(1),
    )(xb, w1b)
    ss1 = _finalize(ps1, g_pw1, b_pw1, m)

    z2, ps2 = pl.pallas_call(
        _dw_kernel, grid=(n,),
        in_specs=[pl.BlockSpec((None, h, w, cm), lambda i: (i, 0, 0, 0)),
                  pl.BlockSpec((2, cm), lambda i: (0, 0)),
                  pl.BlockSpec((9, cm), lambda i: (0, 0))],
        out_specs=[pl.BlockSpec((None, h, w, cm), lambda i: (i, 0, 0, 0)),
                   pl.BlockSpec((None, 2, cm), lambda i: (i, 0, 0))],
        out_shape=[jax.ShapeDtypeStruct((n, h, w, cm), jnp.bfloat16),
                   jax.ShapeDtypeStruct((n, 2, cm), jnp.float32)],
        compiler_params=_parallel(1),
    )(z1.reshape(n, h, w, cm), ss1, w9)
    ss2 = _finalize(ps2, g_dw, b_dw, m)

    z3, ps3 = pl.pallas_call(
        _pw2_kernel, grid=(nt,),
        in_specs=[pl.BlockSpec((bm, cm), lambda i: (i, 0)),
                  pl.BlockSpec((2, cm), lambda i: (0, 0)),
                  pl.BlockSpec((cm, co), lambda i: (0, 0))],
        out_specs=[pl.BlockSpec((bm, co), lambda i: (i, 0)),
                   pl.BlockSpec((None, 2, co), lambda i: (i, 0, 0))],
        out_shape=[jax.ShapeDtypeStruct((m, co), jnp.float32),
                   jax.ShapeDtypeStruct((nt, 2, co), jnp.float32)],
        compiler_params=_parallel(1),
    )(z2.reshape(m, cm), ss2, w2b)
    ss3 = _finalize(ps3, g_pw2, b_pw2, m)

    out2d = pl.pallas_call(
        _bn_res_kernel, grid=(nt,),
        in_specs=[pl.BlockSpec((bm, co), lambda i: (i, 0)),
                  pl.BlockSpec((2, co), lambda i: (0, 0)),
                  pl.BlockSpec((bm, co), lambda i: (i, 0))],
        out_specs=pl.BlockSpec((bm, co), lambda i: (i, 0)),
        out_shape=jax.ShapeDtypeStruct((m, co), jnp.float32),
        compiler_params=_parallel(1),
    )(z3, ss3, x2d)

    out = out2d.reshape(n, h, w, co)
    return jnp.transpose(out, (0, 3, 1, 2))


# speedup vs baseline: 3.8356x; 1.1414x over previous
import functools

import jax
import jax.numpy as jnp
from jax import lax
from jax.experimental import pallas as pl
from jax.experimental.pallas import tpu as pltpu

EPS = 1e-5
BM = 1024


def _colsums(ps_ref, z):
    ones = jnp.ones((1, z.shape[0]), jnp.float32)
    ps_ref[0:1, :] = jnp.dot(ones, z, preferred_element_type=jnp.float32)
    ps_ref[1:2, :] = jnp.dot(ones, z * z, preferred_element_type=jnp.float32)


def _pw1_kernel(x_ref, w_ref, z_ref, ps_ref):
    z = jnp.dot(x_ref[...].astype(jnp.bfloat16), w_ref[...],
                preferred_element_type=jnp.float32)
    z_ref[...] = z.astype(jnp.bfloat16)
    _colsums(ps_ref, z)


def _dw_kernel(z_ref, ss_ref, w_ref, o_ref, ps_ref):
    h, w, cm = z_ref.shape
    nchunks = 4
    hh = h // nchunks
    s = ss_ref[0:1, :]
    b = ss_ref[1:2, :]
    wt = w_ref[...]
    psum, psq = None, None
    ones = jnp.ones((1, hh * w), jnp.float32)
    for chunk in range(nchunks):
        r0 = chunk * hh
        lo = max(r0 - 1, 0)
        hi = min(r0 + hh + 1, h)
        zs = z_ref[lo:hi]
        y = jnp.maximum(zs.astype(jnp.float32) * s + b, 0.0)
        yp = jnp.pad(y, ((lo - (r0 - 1), (r0 + hh + 1) - hi), (1, 1), (0, 0)))
        acc = jnp.zeros((hh, w, cm), jnp.float32)
        for kw in range(3):
            t = yp[0:hh, :, :] * wt[kw:kw + 1, :]
            t = t + yp[1:hh + 1, :, :] * wt[3 + kw:4 + kw, :]
            t = t + yp[2:hh + 2, :, :] * wt[6 + kw:7 + kw, :]
            acc = acc + t[:, kw:kw + w, :]
        o_ref[r0:r0 + hh] = acc.astype(jnp.bfloat16)
        a2 = acc.reshape(hh * w, cm)
        ssum = jnp.dot(ones, a2, preferred_element_type=jnp.float32)
        ssq = jnp.dot(ones, a2 * a2, preferred_element_type=jnp.float32)
        psum = ssum if psum is None else psum + ssum
        psq = ssq if psq is None else psq + ssq
    ps_ref[0:1, :] = psum
    ps_ref[1:2, :] = psq


def _pw2_kernel(z_ref, ss_ref, w_ref, z3_ref, ps_ref):
    y = jnp.maximum(z_ref[...].astype(jnp.float32) * ss_ref[0:1, :]
                    + ss_ref[1:2, :], 0.0)
    z3 = jnp.dot(y.astype(jnp.bfloat16), w_ref[...],
                 preferred_element_type=jnp.float32)
    z3_ref[...] = z3
    _colsums(ps_ref, z3)


def _bn_res_kernel(z_ref, ss_ref, r_ref, o_ref):
    o_ref[...] = (jnp.maximum(z_ref[...] * ss_ref[0:1, :] + ss_ref[1:2, :],
                              0.0) + r_ref[...])


def _finalize(ps, g, b, count):
    s = ps.sum(axis=0)
    mu = s[0:1, :] / count
    var = s[1:2, :] / count - mu * mu
    scale = g * lax.rsqrt(var + EPS)
    shift = b - mu * scale
    return jnp.concatenate([scale, shift], axis=0)


def _parallel(n):
    return pltpu.CompilerParams(dimension_semantics=("parallel",) * n)


def kernel(x, w_pw1, g_pw1, b_pw1, w_dw, g_dw, b_dw, w_pw2, g_pw2, b_pw2):
    n, c, h, w = x.shape
    cm = w_pw1.shape[1]
    co = w_pw2.shape[1]
    m = n * h * w
    bm = BM if m % BM == 0 else w * (h // 2)
    nt = m // bm

    x_nhwc = jnp.transpose(x, (0, 2, 3, 1))
    x2d = x_nhwc.reshape(m, c)
    w1b = w_pw1.astype(jnp.bfloat16)
    w2b = w_pw2.astype(jnp.bfloat16)
    w9 = w_dw.reshape(9, cm)

    z1, ps1 = pl.pallas_call(
        _pw1_kernel, grid=(nt,),
        in_specs=[pl.BlockSpec((bm, c), lambda i: (i, 0)),
                  pl.BlockSpec((c, cm), lambda i: (0, 0))],
        out_specs=[pl.BlockSpec((bm, cm), lambda i: (i, 0)),
                   pl.BlockSpec((None, 2, cm), lambda i: (i, 0, 0))],
        out_shape=[jax.ShapeDtypeStruct((m, cm), jnp.bfloat16),
                   jax.ShapeDtypeStruct((nt, 2, cm), jnp.float32)],
        compiler_params=_parallel(1),
    )(x2d, w1b)
    ss1 = _finalize(ps1, g_pw1, b_pw1, m)

    z2, ps2 = pl.pallas_call(
        _dw_kernel, grid=(n,),
        in_specs=[pl.BlockSpec((None, h, w, cm), lambda i: (i, 0, 0, 0)),
                  pl.BlockSpec((2, cm), lambda i: (0, 0)),
                  pl.BlockSpec((9, cm), lambda i: (0, 0))],
        out_specs=[pl.BlockSpec((None, h, w, cm), lambda i: (i, 0, 0, 0)),
                   pl.BlockSpec((None, 2, cm), lambda i: (i, 0, 0))],
        out_shape=[jax.ShapeDtypeStruct((n, h, w, cm), jnp.bfloat16),
                   jax.ShapeDtypeStruct((n, 2, cm), jnp.float32)],
        compiler_params=_parallel(1),
    )(z1.reshape(n, h, w, cm), ss1, w9)
    ss2 = _finalize(ps2, g_dw, b_dw, m)

    z3, ps3 = pl.pallas_call(
        _pw2_kernel, grid=(nt,),
        in_specs=[pl.BlockSpec((bm, cm), lambda i: (i, 0)),
                  pl.BlockSpec((2, cm), lambda i: (0, 0)),
                  pl.BlockSpec((cm, co), lambda i: (0, 0))],
        out_specs=[pl.BlockSpec((bm, co), lambda i: (i, 0)),
                   pl.BlockSpec((None, 2, co), lambda i: (i, 0, 0))],
        out_shape=[jax.ShapeDtypeStruct((m, co), jnp.float32),
                   jax.ShapeDtypeStruct((nt, 2, co), jnp.float32)],
        compiler_params=_parallel(1),
    )(z2.reshape(m, cm), ss2, w2b)
    ss3 = _finalize(ps3, g_pw2, b_pw2, m)

    out2d = pl.pallas_call(
        _bn_res_kernel, grid=(nt,),
        in_specs=[pl.BlockSpec((bm, co), lambda i: (i, 0)),
                  pl.BlockSpec((2, co), lambda i: (0, 0)),
                  pl.BlockSpec((bm, co), lambda i: (i, 0))],
        out_specs=pl.BlockSpec((bm, co), lambda i: (i, 0)),
        out_shape=jax.ShapeDtypeStruct((m, co), jnp.float32),
        compiler_params=_parallel(1),
    )(z3, ss3, x2d)

    out = out2d.reshape(n, h, w, co)
    return jnp.transpose(out, (0, 3, 1, 2))


# speedup vs baseline: 4.8807x; 1.2725x over previous
import functools

import jax
import jax.numpy as jnp
from jax import lax
from jax.experimental import pallas as pl
from jax.experimental.pallas import tpu as pltpu

EPS = 1e-5
BM = 1024


def _colsums(ps_ref, z):
    ones = jnp.ones((1, z.shape[0]), jnp.float32)
    ps_ref[0:1, :] = jnp.dot(ones, z, preferred_element_type=jnp.float32)
    ps_ref[1:2, :] = jnp.dot(ones, z * z, preferred_element_type=jnp.float32)


def _pw1_kernel(x_ref, w_ref, z_ref, ps_ref):
    xb = x_ref[...].astype(jnp.bfloat16)
    z = lax.dot_general(xb, w_ref[...], (((0,), (0,)), ((), ())),
                        preferred_element_type=jnp.float32)
    z_ref[...] = z.astype(jnp.bfloat16)
    _colsums(ps_ref, z)


def _dw_kernel(z_ref, ss_ref, w_ref, o_ref, ps_ref):
    h, w, cm = z_ref.shape
    nchunks = 4
    hh = h // nchunks
    s = ss_ref[0:1, :]
    b = ss_ref[1:2, :]
    wt = w_ref[...]
    psum, psq = None, None
    ones = jnp.ones((1, hh * w), jnp.float32)
    for chunk in range(nchunks):
        r0 = chunk * hh
        lo = max(r0 - 1, 0)
        hi = min(r0 + hh + 1, h)
        zs = z_ref[lo:hi]
        y = jnp.maximum(zs.astype(jnp.float32) * s + b, 0.0)
        yp = jnp.pad(y, ((lo - (r0 - 1), (r0 + hh + 1) - hi), (1, 1), (0, 0)))
        acc = jnp.zeros((hh, w, cm), jnp.float32)
        for kw in range(3):
            t = yp[0:hh, :, :] * wt[kw:kw + 1, :]
            t = t + yp[1:hh + 1, :, :] * wt[3 + kw:4 + kw, :]
            t = t + yp[2:hh + 2, :, :] * wt[6 + kw:7 + kw, :]
            acc = acc + t[:, kw:kw + w, :]
        o_ref[r0:r0 + hh] = acc.astype(jnp.bfloat16)
        a2 = acc.reshape(hh * w, cm)
        ssum = jnp.dot(ones, a2, preferred_element_type=jnp.float32)
        ssq = jnp.dot(ones, a2 * a2, preferred_element_type=jnp.float32)
        psum = ssum if psum is None else psum + ssum
        psq = ssq if psq is None else psq + ssq
    ps_ref[0:1, :] = psum
    ps_ref[1:2, :] = psq


def _pw2_kernel(z_ref, ss_ref, w_ref, eye_ref, zt_ref, ps_ref):
    y = jnp.maximum(z_ref[...].astype(jnp.float32) * ss_ref[0:1, :]
                    + ss_ref[1:2, :], 0.0)
    z3 = jnp.dot(y.astype(jnp.bfloat16), w_ref[...],
                 preferred_element_type=jnp.float32)
    zt_ref[...] = lax.dot_general(eye_ref[...], z3, (((1,), (1,)), ((), ())),
                                  preferred_element_type=jnp.float32)
    _colsums(ps_ref, z3)


def _bn_res_kernel(z_ref, ss_ref, r_ref, o_ref):
    s = ss_ref[:, 0:1]
    b = ss_ref[:, 1:2]
    o_ref[...] = jnp.maximum(z_ref[...] * s + b, 0.0) + r_ref[...]


def _finalize(ps, g, b, count):
    s = ps.sum(axis=0)
    mu = s[0:1, :] / count
    var = s[1:2, :] / count - mu * mu
    scale = g * lax.rsqrt(var + EPS)
    shift = b - mu * scale
    return jnp.concatenate([scale, shift], axis=0)


def _parallel(n):
    return pltpu.CompilerParams(dimension_semantics=("parallel",) * n)


def kernel(x, w_pw1, g_pw1, b_pw1, w_dw, g_dw, b_dw, w_pw2, g_pw2, b_pw2):
    n, c, h, w = x.shape
    cm = w_pw1.shape[1]
    co = w_pw2.shape[1]
    hw = h * w
    m = n * hw
    bm = BM if m % BM == 0 else hw
    nt = m // bm

    x3 = x.reshape(n, c, hw)
    w1b = w_pw1.astype(jnp.bfloat16)
    w2b = w_pw2.astype(jnp.bfloat16)
    w9 = w_dw.reshape(9, cm)
    eye = jnp.eye(co, dtype=jnp.float32)

    z1, ps1 = pl.pallas_call(
        _pw1_kernel, grid=(n,),
        in_specs=[pl.BlockSpec((None, c, hw), lambda i: (i, 0, 0)),
                  pl.BlockSpec((c, cm), lambda i: (0, 0))],
        out_specs=[pl.BlockSpec((None, hw, cm), lambda i: (i, 0, 0)),
                   pl.BlockSpec((None, 2, cm), lambda i: (i, 0, 0))],
        out_shape=[jax.ShapeDtypeStruct((n, hw, cm), jnp.bfloat16),
                   jax.ShapeDtypeStruct((n, 2, cm), jnp.float32)],
        compiler_params=_parallel(1),
    )(x3, w1b)
    ss1 = _finalize(ps1, g_pw1, b_pw1, m)

    z2, ps2 = pl.pallas_call(
        _dw_kernel, grid=(n,),
        in_specs=[pl.BlockSpec((None, h, w, cm), lambda i: (i, 0, 0, 0)),
                  pl.BlockSpec((2, cm), lambda i: (0, 0)),
                  pl.BlockSpec((9, cm), lambda i: (0, 0))],
        out_specs=[pl.BlockSpec((None, h, w, cm), lambda i: (i, 0, 0, 0)),
                   pl.BlockSpec((None, 2, cm), lambda i: (i, 0, 0))],
        out_shape=[jax.ShapeDtypeStruct((n, h, w, cm), jnp.bfloat16),
                   jax.ShapeDtypeStruct((n, 2, cm), jnp.float32)],
        compiler_params=_parallel(1),
    )(z1.reshape(n, h, w, cm), ss1, w9)
    ss2 = _finalize(ps2, g_dw, b_dw, m)

    z3t, ps3 = pl.pallas_call(
        _pw2_kernel, grid=(nt,),
        in_specs=[pl.BlockSpec((bm, cm), lambda i: (i, 0)),
                  pl.BlockSpec((2, cm), lambda i: (0, 0)),
                  pl.BlockSpec((cm, co), lambda i: (0, 0)),
                  pl.BlockSpec((co, co), lambda i: (0, 0))],
        out_specs=[pl.BlockSpec((co, bm), lambda i: (0, i)),
                   pl.BlockSpec((None, 2, co), lambda i: (i, 0, 0))],
        out_shape=[jax.ShapeDtypeStruct((co, m), jnp.float32),
                   jax.ShapeDtypeStruct((nt, 2, co), jnp.float32)],
        compiler_params=_parallel(1),
    )(z2.reshape(m, cm), ss2, w2b, eye)
    ss3 = _finalize(ps3, g_pw2, b_pw2, m)

    out3 = pl.pallas_call(
        _bn_res_kernel, grid=(n,),
        in_specs=[pl.BlockSpec((co, hw), lambda i: (0, i)),
                  pl.BlockSpec((co, 2), lambda i: (0, 0)),
                  pl.BlockSpec((None, c, hw), lambda i: (i, 0, 0))],
        out_specs=pl.BlockSpec((None, co, hw), lambda i: (i, 0, 0)),
        out_shape=jax.ShapeDtypeStruct((n, co, hw), jnp.float32),
        compiler_params=_parallel(1),
    )(z3t, jnp.transpose(ss3), x3)

    return out3.reshape(n, co, h, w)


# speedup vs baseline: 5.7661x; 1.1814x over previous
import functools

import jax
import jax.numpy as jnp
from jax import lax
from jax.experimental import pallas as pl
from jax.experimental.pallas import tpu as pltpu

EPS = 1e-5
BM = 2048


def _colsums(ps_ref, z):
    ones = jnp.ones((1, z.shape[0]), jnp.float32)
    ps_ref[0:1, :] = jnp.dot(ones, z, preferred_element_type=jnp.float32)
    ps_ref[1:2, :] = jnp.dot(ones, z * z, preferred_element_type=jnp.float32)


def _gram_kernel(x_ref, g_ref, xs_ref):
    xb = x_ref[...].astype(jnp.bfloat16)
    g_ref[...] = lax.dot_general(xb, xb, (((1,), (1,)), ((), ())),
                                 preferred_element_type=jnp.float32)
    xs_ref[...] = jnp.sum(x_ref[...], axis=1, keepdims=True)


def _dw_kernel(x_ref, ss_ref, w1_ref, w_ref, o_ref, ps_ref, *, h, w):
    cm = w1_ref.shape[1]
    nchunks = 4
    hh = h // nchunks
    s = ss_ref[0:1, :]
    b = ss_ref[1:2, :]
    wt = w_ref[...]
    psum, psq = None, None
    ones = jnp.ones((1, hh * w), jnp.float32)
    for chunk in range(nchunks):
        r0 = chunk * hh
        lo = max(r0 - 1, 0)
        hi = min(r0 + hh + 1, h)
        xb = x_ref[:, lo * w:hi * w].astype(jnp.bfloat16)
        zc = lax.dot_general(xb, w1_ref[...], (((0,), (0,)), ((), ())),
                             preferred_element_type=jnp.float32)
        y = jnp.maximum(zc * s + b, 0.0).reshape(hi - lo, w, cm)
        yp = jnp.pad(y, ((lo - (r0 - 1), (r0 + hh + 1) - hi), (1, 1), (0, 0)))
        acc = jnp.zeros((hh, w, cm), jnp.float32)
        for kw in range(3):
            t = yp[0:hh, :, :] * wt[kw:kw + 1, :]
            t = t + yp[1:hh + 1, :, :] * wt[3 + kw:4 + kw, :]
            t = t + yp[2:hh + 2, :, :] * wt[6 + kw:7 + kw, :]
            acc = acc + t[:, kw:kw + w, :]
        o_ref[r0:r0 + hh] = acc.astype(jnp.bfloat16)
        a2 = acc.reshape(hh * w, cm)
        ssum = jnp.dot(ones, a2, preferred_element_type=jnp.float32)
        ssq = jnp.dot(ones, a2 * a2, preferred_element_type=jnp.float32)
        psum = ssum if psum is None else psum + ssum
        psq = ssq if psq is None else psq + ssq
    ps_ref[0:1, :] = psum
    ps_ref[1:2, :] = psq


def _pw2_kernel(z_ref, ss_ref, w_ref, eye_ref, zt_ref, ps_ref):
    y = jnp.maximum(z_ref[...].astype(jnp.float32) * ss_ref[0:1, :]
                    + ss_ref[1:2, :], 0.0)
    z3 = jnp.dot(y.astype(jnp.bfloat16), w_ref[...],
                 preferred_element_type=jnp.float32)
    zt_ref[...] = lax.dot_general(eye_ref[...], z3, (((1,), (1,)), ((), ())),
                                  preferred_element_type=jnp.float32)
    _colsums(ps_ref, z3)


def _bn_res_kernel(z_ref, ss_ref, r_ref, o_ref):
    s = ss_ref[:, 0:1]
    b = ss_ref[:, 1:2]
    o_ref[...] = jnp.maximum(z_ref[...] * s + b, 0.0) + r_ref[...]


def _fold(total_sum, total_sumsq, g, b, count):
    mu = total_sum / count
    var = total_sumsq / count - mu * mu
    scale = g * lax.rsqrt(var + EPS)
    shift = b - mu * scale
    return jnp.concatenate([scale, shift], axis=0)


def _finalize(ps, g, b, count):
    s = ps.sum(axis=0)
    return _fold(s[0:1, :], s[1:2, :], g, b, count)


def _parallel(n):
    return pltpu.CompilerParams(dimension_semantics=("parallel",) * n)


def kernel(x, w_pw1, g_pw1, b_pw1, w_dw, g_dw, b_dw, w_pw2, g_pw2, b_pw2):
    n, c, h, w = x.shape
    cm = w_pw1.shape[1]
    co = w_pw2.shape[1]
    hw = h * w
    m = n * hw
    bm = BM if m % BM == 0 else hw
    nt = m // bm

    x3 = x.reshape(n, c, hw)
    w1b = w_pw1.astype(jnp.bfloat16)
    w2b = w_pw2.astype(jnp.bfloat16)
    w9 = w_dw.reshape(9, cm)
    eye = jnp.eye(co, dtype=jnp.float32)

    gram, xs = pl.pallas_call(
        _gram_kernel, grid=(n,),
        in_specs=[pl.BlockSpec((None, c, hw), lambda i: (i, 0, 0))],
        out_specs=[pl.BlockSpec((None, c, c), lambda i: (i, 0, 0)),
                   pl.BlockSpec((None, c, 1), lambda i: (i, 0, 0))],
        out_shape=[jax.ShapeDtypeStruct((n, c, c), jnp.float32),
                   jax.ShapeDtypeStruct((n, c, 1), jnp.float32)],
        compiler_params=_parallel(1),
    )(x3)
    w1f = w1b.astype(jnp.float32)
    g_sum = gram.sum(axis=0)
    sum1 = jnp.dot(xs.sum(axis=0).reshape(1, c), w1f)
    sumsq1 = (jnp.dot(g_sum, w1f) * w1f).sum(axis=0, keepdims=True)
    ss1 = _fold(sum1, sumsq1, g_pw1, b_pw1, m)

    z2, ps2 = pl.pallas_call(
        functools.partial(_dw_kernel, h=h, w=w), grid=(n,),
        in_specs=[pl.BlockSpec((None, c, hw), lambda i: (i, 0, 0)),
                  pl.BlockSpec((2, cm), lambda i: (0, 0)),
                  pl.BlockSpec((c, cm), lambda i: (0, 0)),
                  pl.BlockSpec((9, cm), lambda i: (0, 0))],
        out_specs=[pl.BlockSpec((None, h, w, cm), lambda i: (i, 0, 0, 0)),
                   pl.BlockSpec((None, 2, cm), lambda i: (i, 0, 0))],
        out_shape=[jax.ShapeDtypeStruct((n, h, w, cm), jnp.bfloat16),
                   jax.ShapeDtypeStruct((n, 2, cm), jnp.float32)],
        compiler_params=_parallel(1),
    )(x3, ss1, w1b, w9)
    ss2 = _finalize(ps2, g_dw, b_dw, m)

    z3t, ps3 = pl.pallas_call(
        _pw2_kernel, grid=(nt,),
        in_specs=[pl.BlockSpec((bm, cm), lambda i: (i, 0)),
                  pl.BlockSpec((2, cm), lambda i: (0, 0)),
                  pl.BlockSpec((cm, co), lambda i: (0, 0)),
                  pl.BlockSpec((co, co), lambda i: (0, 0))],
        out_specs=[pl.BlockSpec((co, bm), lambda i: (0, i)),
                   pl.BlockSpec((None, 2, co), lambda i: (i, 0, 0))],
        out_shape=[jax.ShapeDtypeStruct((co, m), jnp.float32),
                   jax.ShapeDtypeStruct((nt, 2, co), jnp.float32)],
        compiler_params=_parallel(1),
    )(z2.reshape(m, cm), ss2, w2b, eye)
    ss3 = _finalize(ps3, g_pw2, b_pw2, m)

    out3 = pl.pallas_call(
        _bn_res_kernel, grid=(n,),
        in_specs=[pl.BlockSpec((co, hw), lambda i: (0, i)),
                  pl.BlockSpec((co, 2), lambda i: (0, 0)),
                  pl.BlockSpec((None, c, hw), lambda i: (i, 0, 0))],
        out_specs=pl.BlockSpec((None, co, hw), lambda i: (i, 0, 0)),
        out_shape=jax.ShapeDtypeStruct((n, co, hw), jnp.float32),
        compiler_params=_parallel(1),
    )(z3t, jnp.transpose(ss3), x3)

    return out3.reshape(n, co, h, w)


# speedup vs baseline: 5.8518x; 1.0148x over previous
import functools

import jax
import jax.numpy as jnp
from jax import lax
from jax.experimental import pallas as pl
from jax.experimental.pallas import tpu as pltpu

EPS = 1e-5
BM = 2048


def _fold_ss(ssum, ssq, gb, count):
    mu = ssum / count
    var = ssq / count - mu * mu
    scale = gb[0:1, :] * lax.rsqrt(var + EPS)
    shift = gb[1:2, :] - mu * scale
    return scale, shift


def _gram_kernel(x_ref, w1_ref, gb_ref, ss_ref, acc_ref, *, count):
    i = pl.program_id(0)
    c = x_ref.shape[0]
    xb = x_ref[...].astype(jnp.bfloat16)
    g = lax.dot_general(xb, xb, (((1,), (1,)), ((), ())),
                        preferred_element_type=jnp.float32)
    xs = jnp.sum(x_ref[...], axis=1, keepdims=True)
    part = jnp.concatenate([g, xs], axis=1)

    @pl.when(i == 0)
    def _():
        acc_ref[...] = part

    @pl.when(i > 0)
    def _():
        acc_ref[...] = acc_ref[...] + part

    @pl.when(i == pl.num_programs(0) - 1)
    def _():
        w1f = w1_ref[...].astype(jnp.float32)
        gg = acc_ref[:, 0:c]
        sum1 = lax.dot_general(acc_ref[:, c:c + 1], w1f, (((0,), (0,)), ((), ())),
                               preferred_element_type=jnp.float32)
        ones = jnp.ones((1, c), jnp.float32)
        sumsq1 = jnp.dot(ones, jnp.dot(gg, w1f,
                                       preferred_element_type=jnp.float32) * w1f,
                         preferred_element_type=jnp.float32)
        scale, shift = _fold_ss(sum1, sumsq1, gb_ref[...], count)
        ss_ref[0:1, :] = scale
        ss_ref[1:2, :] = shift


def _dw_kernel(x_ref, ss_ref, w1_ref, w_ref, gb_ref, o_ref, ss2_ref, acc_ref,
               *, h, w, count):
    i = pl.program_id(0)
    cm = w1_ref.shape[1]
    nchunks = 4
    hh = h // nchunks
    s = ss_ref[0:1, :]
    b = ss_ref[1:2, :]
    wt = w_ref[...]
    psum, psq = None, None
    ones = jnp.ones((1, hh * w), jnp.float32)
    for chunk in range(nchunks):
        r0 = chunk * hh
        lo = max(r0 - 1, 0)
        hi = min(r0 + hh + 1, h)
        xb = x_ref[:, lo * w:hi * w].astype(jnp.bfloat16)
        zc = lax.dot_general(xb, w1_ref[...], (((0,), (0,)), ((), ())),
                             preferred_element_type=jnp.float32)
        y = jnp.maximum(zc * s + b, 0.0).reshape(hi - lo, w, cm)
        yp = jnp.pad(y, ((lo - (r0 - 1), (r0 + hh + 1) - hi), (1, 1), (0, 0)))
        acc = jnp.zeros((hh, w, cm), jnp.float32)
        for kw in range(3):
            t = yp[0:hh, :, :] * wt[kw:kw + 1, :]
            t = t + yp[1:hh + 1, :, :] * wt[3 + kw:4 + kw, :]
            t = t + yp[2:hh + 2, :, :] * wt[6 + kw:7 + kw, :]
            acc = acc + t[:, kw:kw + w, :]
        o_ref[r0:r0 + hh] = acc.astype(jnp.bfloat16)
        a2 = acc.reshape(hh * w, cm)
        ssum = jnp.dot(ones, a2, preferred_element_type=jnp.float32)
        ssq = jnp.dot(ones, a2 * a2, preferred_element_type=jnp.float32)
        psum = ssum if psum is None else psum + ssum
        psq = ssq if psq is None else psq + ssq
    part = jnp.concatenate([psum, psq], axis=0)

    @pl.when(i == 0)
    def _():
        acc_ref[...] = part

    @pl.when(i > 0)
    def _():
        acc_ref[...] = acc_ref[...] + part

    @pl.when(i == pl.num_programs(0) - 1)
    def _():
        scale, shift = _fold_ss(acc_ref[0:1, :], acc_ref[1:2, :],
                                gb_ref[...], count)
        ss2_ref[0:1, :] = scale
        ss2_ref[1:2, :] = shift


def _pw2_kernel(z_ref, ss_ref, w_ref, eye_ref, gb_ref, zt_ref, ss3_ref,
                acc_ref, *, count):
    i = pl.program_id(0)
    y = jnp.maximum(z_ref[...].astype(jnp.float32) * ss_ref[0:1, :]
                    + ss_ref[1:2, :], 0.0)
    z3 = jnp.dot(y.astype(jnp.bfloat16), w_ref[...],
                 preferred_element_type=jnp.float32)
    zt_ref[...] = lax.dot_general(eye_ref[...], z3, (((1,), (1,)), ((), ())),
                                  preferred_element_type=jnp.float32
                                  ).astype(jnp.bfloat16)
    ones = jnp.ones((1, z3.shape[0]), jnp.float32)
    ssum = jnp.dot(ones, z3, preferred_element_type=jnp.float32)
    ssq = jnp.dot(ones, z3 * z3, preferred_element_type=jnp.float32)
    part = jnp.concatenate([ssum, ssq], axis=0)

    @pl.when(i == 0)
    def _():
        acc_ref[...] = part

    @pl.when(i > 0)
    def _():
        acc_ref[...] = acc_ref[...] + part

    @pl.when(i == pl.num_programs(0) - 1)
    def _():
        scale, shift = _fold_ss(acc_ref[0:1, :], acc_ref[1:2, :],
                                gb_ref[...], count)
        st = jnp.concatenate([scale, shift], axis=0)
        ss3_ref[...] = lax.dot_general(eye_ref[...], st, (((1,), (1,)), ((), ())),
                                       preferred_element_type=jnp.float32)


def _bn_res_kernel(z_ref, ss_ref, r_ref, o_ref):
    s = ss_ref[:, 0:1]
    b = ss_ref[:, 1:2]
    o_ref[...] = (jnp.maximum(z_ref[...].astype(jnp.float32) * s + b, 0.0)
                  + r_ref[...])


def kernel(x, w_pw1, g_pw1, b_pw1, w_dw, g_dw, b_dw, w_pw2, g_pw2, b_pw2):
    n, c, h, w = x.shape
    cm = w_pw1.shape[1]
    co = w_pw2.shape[1]
    hw = h * w
    m = n * hw
    bm = BM if m % BM == 0 else hw
    nt = m // bm
    fm = float(m)

    x3 = x.reshape(n, c, hw)
    w1b = w_pw1.astype(jnp.bfloat16)
    w2b = w_pw2.astype(jnp.bfloat16)
    w9 = w_dw.reshape(9, cm)
    eye = jnp.eye(co, dtype=jnp.float32)
    gb1 = jnp.concatenate([g_pw1, b_pw1], axis=0)
    gb2 = jnp.concatenate([g_dw, b_dw], axis=0)
    gb3 = jnp.concatenate([g_pw2, b_pw2], axis=0)

    arb = pltpu.CompilerParams(dimension_semantics=("arbitrary",))

    ss1 = pl.pallas_call(
        functools.partial(_gram_kernel, count=fm), grid=(n,),
        in_specs=[pl.BlockSpec((None, c, hw), lambda i: (i, 0, 0)),
                  pl.BlockSpec((c, cm), lambda i: (0, 0)),
                  pl.BlockSpec((2, cm), lambda i: (0, 0))],
        out_specs=pl.BlockSpec((2, cm), lambda i: (0, 0)),
        out_shape=jax.ShapeDtypeStruct((2, cm), jnp.float32),
        scratch_shapes=[pltpu.VMEM((c, c + 1), jnp.float32)],
        compiler_params=arb,
    )(x3, w1b, gb1)

    z2, ss2 = pl.pallas_call(
        functools.partial(_dw_kernel, h=h, w=w, count=fm), grid=(n,),
        in_specs=[pl.BlockSpec((None, c, hw), lambda i: (i, 0, 0)),
                  pl.BlockSpec((2, cm), lambda i: (0, 0)),
                  pl.BlockSpec((c, cm), lambda i: (0, 0)),
                  pl.BlockSpec((9, cm), lambda i: (0, 0)),
                  pl.BlockSpec((2, cm), lambda i: (0, 0))],
        out_specs=[pl.BlockSpec((None, h, w, cm), lambda i: (i, 0, 0, 0)),
                   pl.BlockSpec((2, cm), lambda i: (0, 0))],
        out_shape=[jax.ShapeDtypeStruct((n, h, w, cm), jnp.bfloat16),
                   jax.ShapeDtypeStruct((2, cm), jnp.float32)],
        scratch_shapes=[pltpu.VMEM((2, cm), jnp.float32)],
        compiler_params=arb,
    )(x3, ss1, w1b, w9, gb2)

    z3t, ss3t = pl.pallas_call(
        functools.partial(_pw2_kernel, count=fm), grid=(nt,),
        in_specs=[pl.BlockSpec((bm, cm), lambda i: (i, 0)),
                  pl.BlockSpec((2, cm), lambda i: (0, 0)),
                  pl.BlockSpec((cm, co), lambda i: (0, 0)),
                  pl.BlockSpec((co, co), lambda i: (0, 0)),
                  pl.BlockSpec((2, co), lambda i: (0, 0))],
        out_specs=[pl.BlockSpec((co, bm), lambda i: (0, i)),
                   pl.BlockSpec((co, 2), lambda i: (0, 0))],
        out_shape=[jax.ShapeDtypeStruct((co, m), jnp.bfloat16),
                   jax.ShapeDtypeStruct((co, 2), jnp.float32)],
        scratch_shapes=[pltpu.VMEM((2, co), jnp.float32)],
        compiler_params=arb,
    )(z2.reshape(m, cm), ss2, w2b, eye, gb3)

    out3 = pl.pallas_call(
        _bn_res_kernel, grid=(n,),
        in_specs=[pl.BlockSpec((co, hw), lambda i: (0, i)),
                  pl.BlockSpec((co, 2), lambda i: (0, 0)),
                  pl.BlockSpec((None, c, hw), lambda i: (i, 0, 0))],
        out_specs=pl.BlockSpec((None, co, hw), lambda i: (i, 0, 0)),
        out_shape=jax.ShapeDtypeStruct((n, co, hw), jnp.float32),
        compiler_params=pltpu.CompilerParams(
            dimension_semantics=("parallel",)),
    )(z3t, ss3t, x3)

    return out3.reshape(n, co, h, w)


# speedup vs baseline: 6.1572x; 1.0522x over previous
import functools

import jax
import jax.numpy as jnp
from jax import lax
from jax.experimental import pallas as pl
from jax.experimental.pallas import tpu as pltpu

EPS = 1e-5
BM = 4096


def _fold_ss(ssum, ssq, gb, count):
    mu = ssum / count
    var = ssq / count - mu * mu
    scale = gb[0:1, :] * lax.rsqrt(var + EPS)
    shift = gb[1:2, :] - mu * scale
    return scale, shift


def _gram_kernel(x_ref, w1_ref, gb_ref, ss_ref, acc_ref, *, count):
    i = pl.program_id(0)
    c = x_ref.shape[0]
    xb = x_ref[...].astype(jnp.bfloat16)
    g = lax.dot_general(xb, xb, (((1,), (1,)), ((), ())),
                        preferred_element_type=jnp.float32)
    xs = jnp.sum(x_ref[...], axis=1, keepdims=True)
    part = jnp.concatenate([g, xs], axis=1)

    @pl.when(i == 0)
    def _():
        acc_ref[...] = part

    @pl.when(i > 0)
    def _():
        acc_ref[...] = acc_ref[...] + part

    @pl.when(i == pl.num_programs(0) - 1)
    def _():
        w1f = w1_ref[...].astype(jnp.float32)
        gg = acc_ref[:, 0:c]
        sum1 = lax.dot_general(acc_ref[:, c:c + 1], w1f, (((0,), (0,)), ((), ())),
                               preferred_element_type=jnp.float32)
        ones = jnp.ones((1, c), jnp.float32)
        sumsq1 = jnp.dot(ones, jnp.dot(gg, w1f,
                                       preferred_element_type=jnp.float32) * w1f,
                         preferred_element_type=jnp.float32)
        scale, shift = _fold_ss(sum1, sumsq1, gb_ref[...], count)
        ss_ref[0:1, :] = scale
        ss_ref[1:2, :] = shift


def _dw_kernel(x_ref, ss_ref, w1_ref, w_ref, gb_ref, o_ref, ss2_ref, acc_ref,
               *, h, w, count):
    i = pl.program_id(0)
    cm = w1_ref.shape[1]
    nchunks = 3 if h % 3 == 0 else (2 if h % 2 == 0 else 1)
    hh = h // nchunks
    s = ss_ref[0:1, :]
    b = ss_ref[1:2, :]
    wt = w_ref[...]
    psum, psq = None, None
    ones = jnp.ones((1, hh * w), jnp.float32)
    for chunk in range(nchunks):
        r0 = chunk * hh
        lo = max(r0 - 1, 0)
        hi = min(r0 + hh + 1, h)
        xb = x_ref[:, lo * w:hi * w].astype(jnp.bfloat16)
        zc = lax.dot_general(xb, w1_ref[...], (((0,), (0,)), ((), ())),
                             preferred_element_type=jnp.float32)
        y = jnp.maximum(zc * s + b, 0.0).reshape(hi - lo, w, cm)
        yp = jnp.pad(y, ((lo - (r0 - 1), (r0 + hh + 1) - hi), (1, 1), (0, 0)))
        acc = None
        for kw in range(3):
            t = yp[0:hh, :, :] * wt[kw:kw + 1, :]
            t = t + yp[1:hh + 1, :, :] * wt[3 + kw:4 + kw, :]
            t = t + yp[2:hh + 2, :, :] * wt[6 + kw:7 + kw, :]
            tc = t[:, kw:kw + w, :]
            acc = tc if acc is None else acc + tc
        o_ref[r0:r0 + hh] = acc.astype(jnp.bfloat16)
        a2 = acc.reshape(hh * w, cm)
        ssum = jnp.dot(ones, a2, preferred_element_type=jnp.float32)
        ssq = jnp.dot(ones, a2 * a2, preferred_element_type=jnp.float32)
        psum = ssum if psum is None else psum + ssum
        psq = ssq if psq is None else psq + ssq
    part = jnp.concatenate([psum, psq], axis=0)

    @pl.when(i == 0)
    def _():
        acc_ref[...] = part

    @pl.when(i > 0)
    def _():
        acc_ref[...] = acc_ref[...] + part

    @pl.when(i == pl.num_programs(0) - 1)
    def _():
        scale, shift = _fold_ss(acc_ref[0:1, :], acc_ref[1:2, :],
                                gb_ref[...], count)
        ss2_ref[0:1, :] = scale
        ss2_ref[1:2, :] = shift


def _pw2_kernel(z_ref, ss_ref, w_ref, eye_ref, gb_ref, zt_ref, ss3_ref,
                acc_ref, *, count):
    i = pl.program_id(0)
    y = jnp.maximum(z_ref[...].astype(jnp.float32) * ss_ref[0:1, :]
                    + ss_ref[1:2, :], 0.0)
    z3 = jnp.dot(y.astype(jnp.bfloat16), w_ref[...],
                 preferred_element_type=jnp.float32)
    zt_ref[...] = lax.dot_general(eye_ref[...], z3, (((1,), (1,)), ((), ())),
                                  preferred_element_type=jnp.float32
                                  ).astype(jnp.bfloat16)
    ones = jnp.ones((1, z3.shape[0]), jnp.float32)
    ssum = jnp.dot(ones, z3, preferred_element_type=jnp.float32)
    ssq = jnp.dot(ones, z3 * z3, preferred_element_type=jnp.float32)
    part = jnp.concatenate([ssum, ssq], axis=0)

    @pl.when(i == 0)
    def _():
        acc_ref[...] = part

    @pl.when(i > 0)
    def _():
        acc_ref[...] = acc_ref[...] + part

    @pl.when(i == pl.num_programs(0) - 1)
    def _():
        scale, shift = _fold_ss(acc_ref[0:1, :], acc_ref[1:2, :],
                                gb_ref[...], count)
        st = jnp.concatenate([scale, shift], axis=0)
        ss3_ref[...] = lax.dot_general(eye_ref[...], st, (((1,), (1,)), ((), ())),
                                       preferred_element_type=jnp.float32)


def _bn_res_kernel(z_ref, ss_ref, r_ref, o_ref):
    s = ss_ref[:, 0:1]
    b = ss_ref[:, 1:2]
    o_ref[...] = (jnp.maximum(z_ref[...].astype(jnp.float32) * s + b, 0.0)
                  + r_ref[...])


def kernel(x, w_pw1, g_pw1, b_pw1, w_dw, g_dw, b_dw, w_pw2, g_pw2, b_pw2):
    n, c, h, w = x.shape
    cm = w_pw1.shape[1]
    co = w_pw2.shape[1]
    hw = h * w
    m = n * hw
    bm = BM if m % BM == 0 else hw
    nt = m // bm
    fm = float(m)

    x3 = x.reshape(n, c, hw)
    w1b = w_pw1.astype(jnp.bfloat16)
    w2b = w_pw2.astype(jnp.bfloat16)
    w9 = w_dw.reshape(9, cm)
    eye = jnp.eye(co, dtype=jnp.float32)
    gb1 = jnp.concatenate([g_pw1, b_pw1], axis=0)
    gb2 = jnp.concatenate([g_dw, b_dw], axis=0)
    gb3 = jnp.concatenate([g_pw2, b_pw2], axis=0)

    arb = pltpu.CompilerParams(dimension_semantics=("arbitrary",))

    ss1 = pl.pallas_call(
        functools.partial(_gram_kernel, count=fm), grid=(n,),
        in_specs=[pl.BlockSpec((None, c, hw), lambda i: (i, 0, 0)),
                  pl.BlockSpec((c, cm), lambda i: (0, 0)),
                  pl.BlockSpec((2, cm), lambda i: (0, 0))],
        out_specs=pl.BlockSpec((2, cm), lambda i: (0, 0)),
        out_shape=jax.ShapeDtypeStruct((2, cm), jnp.float32),
        scratch_shapes=[pltpu.VMEM((c, c + 1), jnp.float32)],
        compiler_params=arb,
    )(x3, w1b, gb1)

    z2, ss2 = pl.pallas_call(
        functools.partial(_dw_kernel, h=h, w=w, count=fm), grid=(n,),
        in_specs=[pl.BlockSpec((None, c, hw), lambda i: (i, 0, 0)),
                  pl.BlockSpec((2, cm), lambda i: (0, 0)),
                  pl.BlockSpec((c, cm), lambda i: (0, 0)),
                  pl.BlockSpec((9, cm), lambda i: (0, 0)),
                  pl.BlockSpec((2, cm), lambda i: (0, 0))],
        out_specs=[pl.BlockSpec((None, h, w, cm), lambda i: (i, 0, 0, 0)),
                   pl.BlockSpec((2, cm), lambda i: (0, 0))],
        out_shape=[jax.ShapeDtypeStruct((n, h, w, cm), jnp.bfloat16),
                   jax.ShapeDtypeStruct((2, cm), jnp.float32)],
        scratch_shapes=[pltpu.VMEM((2, cm), jnp.float32)],
        compiler_params=arb,
    )(x3, ss1, w1b, w9, gb2)

    z3t, ss3t = pl.pallas_call(
        functools.partial(_pw2_kernel, count=fm), grid=(nt,),
        in_specs=[pl.BlockSpec((bm, cm), lambda i: (i, 0)),
                  pl.BlockSpec((2, cm), lambda i: (0, 0)),
                  pl.BlockSpec((cm, co), lambda i: (0, 0)),
                  pl.BlockSpec((co, co), lambda i: (0, 0)),
                  pl.BlockSpec((2, co), lambda i: (0, 0))],
        out_specs=[pl.BlockSpec((co, bm), lambda i: (0, i)),
                   pl.BlockSpec((co, 2), lambda i: (0, 0))],
        out_shape=[jax.ShapeDtypeStruct((co, m), jnp.bfloat16),
                   jax.ShapeDtypeStruct((co, 2), jnp.float32)],
        scratch_shapes=[pltpu.VMEM((2, co), jnp.float32)],
        compiler_params=arb,
    )(z2.reshape(m, cm), ss2, w2b, eye, gb3)

    out3 = pl.pallas_call(
        _bn_res_kernel, grid=(n,),
        in_specs=[pl.BlockSpec((co, hw), lambda i: (0, i)),
                  pl.BlockSpec((co, 2), lambda i: (0, 0)),
                  pl.BlockSpec((None, c, hw), lambda i: (i, 0, 0))],
        out_specs=pl.BlockSpec((None, co, hw), lambda i: (i, 0, 0)),
        out_shape=jax.ShapeDtypeStruct((n, co, hw), jnp.float32),
        compiler_params=pltpu.CompilerParams(
            dimension_semantics=("parallel",)),
    )(z3t, ss3t, x3)

    return out3.reshape(n, co, h, w)


# speedup vs baseline: 6.2176x; 1.0098x over previous
import functools

import jax
import jax.numpy as jnp
from jax import lax
from jax.experimental import pallas as pl
from jax.experimental.pallas import tpu as pltpu

EPS = 1e-5
BM = 4096


def _fold_ss(ssum, ssq, gb, count):
    mu = ssum / count
    var = ssq / count - mu * mu
    scale = gb[0:1, :] * lax.rsqrt(var + EPS)
    shift = gb[1:2, :] - mu * scale
    return scale, shift


def _gram_kernel(x_ref, w1_ref, gb_ref, ss_ref, acc_ref, *, count):
    i = pl.program_id(0)
    c = x_ref.shape[0]
    xb = x_ref[...].astype(jnp.bfloat16)
    g = lax.dot_general(xb, xb, (((1,), (1,)), ((), ())),
                        preferred_element_type=jnp.float32)
    xs = jnp.sum(x_ref[...], axis=1, keepdims=True)
    part = jnp.concatenate([g, xs], axis=1)

    @pl.when(i == 0)
    def _():
        acc_ref[...] = part

    @pl.when(i > 0)
    def _():
        acc_ref[...] = acc_ref[...] + part

    @pl.when(i == pl.num_programs(0) - 1)
    def _():
        w1f = w1_ref[...].astype(jnp.float32)
        gg = acc_ref[:, 0:c]
        sum1 = lax.dot_general(acc_ref[:, c:c + 1], w1f, (((0,), (0,)), ((), ())),
                               preferred_element_type=jnp.float32)
        ones = jnp.ones((1, c), jnp.float32)
        sumsq1 = jnp.dot(ones, jnp.dot(gg, w1f,
                                       preferred_element_type=jnp.float32) * w1f,
                         preferred_element_type=jnp.float32)
        scale, shift = _fold_ss(sum1, sumsq1, gb_ref[...], count)
        ss_ref[0:1, :] = scale
        ss_ref[1:2, :] = shift


def _dw_kernel(x_ref, ss_ref, w1_ref, w_ref, gb_ref, o_ref, ss2_ref, acc_ref,
               *, h, w, count):
    i = pl.program_id(0)
    cm = w1_ref.shape[1]
    nchunks = 2 if h % 2 == 0 else 1
    hh = h // nchunks
    s = ss_ref[0:1, :]
    b = ss_ref[1:2, :]
    wt = w_ref[...]
    psum, psq = None, None
    ones = jnp.ones((1, hh * w), jnp.float32)
    for chunk in range(nchunks):
        r0 = chunk * hh
        lo = max(r0 - 1, 0)
        hi = min(r0 + hh + 1, h)
        xb = x_ref[:, lo * w:hi * w].astype(jnp.bfloat16)
        zc = lax.dot_general(xb, w1_ref[...], (((0,), (0,)), ((), ())),
                             preferred_element_type=jnp.float32)
        y = jnp.maximum(zc * s + b, 0.0).reshape(hi - lo, w, cm)
        yp = jnp.pad(y, ((lo - (r0 - 1), (r0 + hh + 1) - hi), (1, 1), (0, 0)))
        acc = None
        for kw in range(3):
            t = yp[0:hh, :, :] * wt[kw:kw + 1, :]
            t = t + yp[1:hh + 1, :, :] * wt[3 + kw:4 + kw, :]
            t = t + yp[2:hh + 2, :, :] * wt[6 + kw:7 + kw, :]
            tc = t[:, kw:kw + w, :]
            acc = tc if acc is None else acc + tc
        o_ref[r0:r0 + hh] = acc.astype(jnp.bfloat16)
        a2 = acc.reshape(hh * w, cm)
        ssum = jnp.dot(ones, a2, preferred_element_type=jnp.float32)
        ssq = jnp.dot(ones, a2 * a2, preferred_element_type=jnp.float32)
        psum = ssum if psum is None else psum + ssum
        psq = ssq if psq is None else psq + ssq
    part = jnp.concatenate([psum, psq], axis=0)

    @pl.when(i == 0)
    def _():
        acc_ref[...] = part

    @pl.when(i > 0)
    def _():
        acc_ref[...] = acc_ref[...] + part

    @pl.when(i == pl.num_programs(0) - 1)
    def _():
        scale, shift = _fold_ss(acc_ref[0:1, :], acc_ref[1:2, :],
                                gb_ref[...], count)
        ss2_ref[0:1, :] = scale
        ss2_ref[1:2, :] = shift


def _pw2_kernel(z_ref, ss_ref, w_ref, eye_ref, gb_ref, zt_ref, ss3_ref,
                acc_ref, *, count):
    i = pl.program_id(0)
    y = jnp.maximum(z_ref[...].astype(jnp.float32) * ss_ref[0:1, :]
                    + ss_ref[1:2, :], 0.0)
    z3 = jnp.dot(y.astype(jnp.bfloat16), w_ref[...],
                 preferred_element_type=jnp.float32)
    zt_ref[...] = lax.dot_general(eye_ref[...], z3, (((1,), (1,)), ((), ())),
                                  preferred_element_type=jnp.float32
                                  ).astype(jnp.bfloat16)
    ones = jnp.ones((1, z3.shape[0]), jnp.float32)
    ssum = jnp.dot(ones, z3, preferred_element_type=jnp.float32)
    ssq = jnp.dot(ones, z3 * z3, preferred_element_type=jnp.float32)
    part = jnp.concatenate([ssum, ssq], axis=0)

    @pl.when(i == 0)
    def _():
        acc_ref[...] = part

    @pl.when(i > 0)
    def _():
        acc_ref[...] = acc_ref[...] + part

    @pl.when(i == pl.num_programs(0) - 1)
    def _():
        scale, shift = _fold_ss(acc_ref[0:1, :], acc_ref[1:2, :],
                                gb_ref[...], count)
        st = jnp.concatenate([scale, shift], axis=0)
        ss3_ref[...] = lax.dot_general(eye_ref[...], st, (((1,), (1,)), ((), ())),
                                       preferred_element_type=jnp.float32)


def _bn_res_kernel(z_ref, ss_ref, r_ref, o_ref):
    s = ss_ref[:, 0:1]
    b = ss_ref[:, 1:2]
    o_ref[...] = (jnp.maximum(z_ref[...].astype(jnp.float32) * s + b, 0.0)
                  + r_ref[...])


def kernel(x, w_pw1, g_pw1, b_pw1, w_dw, g_dw, b_dw, w_pw2, g_pw2, b_pw2):
    n, c, h, w = x.shape
    cm = w_pw1.shape[1]
    co = w_pw2.shape[1]
    hw = h * w
    m = n * hw
    bm = BM if m % BM == 0 else hw
    nt = m // bm
    fm = float(m)

    x3 = x.reshape(n, c, hw)
    w1b = w_pw1.astype(jnp.bfloat16)
    w2b = w_pw2.astype(jnp.bfloat16)
    w9 = w_dw.reshape(9, cm)
    eye = jnp.eye(co, dtype=jnp.float32)
    gb1 = jnp.concatenate([g_pw1, b_pw1], axis=0)
    gb2 = jnp.concatenate([g_dw, b_dw], axis=0)
    gb3 = jnp.concatenate([g_pw2, b_pw2], axis=0)

    arb = pltpu.CompilerParams(dimension_semantics=("arbitrary",))

    ss1 = pl.pallas_call(
        functools.partial(_gram_kernel, count=fm), grid=(n,),
        in_specs=[pl.BlockSpec((None, c, hw), lambda i: (i, 0, 0)),
                  pl.BlockSpec((c, cm), lambda i: (0, 0)),
                  pl.BlockSpec((2, cm), lambda i: (0, 0))],
        out_specs=pl.BlockSpec((2, cm), lambda i: (0, 0)),
        out_shape=jax.ShapeDtypeStruct((2, cm), jnp.float32),
        scratch_shapes=[pltpu.VMEM((c, c + 1), jnp.float32)],
        compiler_params=arb,
    )(x3, w1b, gb1)

    z2, ss2 = pl.pallas_call(
        functools.partial(_dw_kernel, h=h, w=w, count=fm), grid=(n,),
        in_specs=[pl.BlockSpec((None, c, hw), lambda i: (i, 0, 0)),
                  pl.BlockSpec((2, cm), lambda i: (0, 0)),
                  pl.BlockSpec((c, cm), lambda i: (0, 0)),
                  pl.BlockSpec((9, cm), lambda i: (0, 0)),
                  pl.BlockSpec((2, cm), lambda i: (0, 0))],
        out_specs=[pl.BlockSpec((None, h, w, cm), lambda i: (i, 0, 0, 0)),
                   pl.BlockSpec((2, cm), lambda i: (0, 0))],
        out_shape=[jax.ShapeDtypeStruct((n, h, w, cm), jnp.bfloat16),
                   jax.ShapeDtypeStruct((2, cm), jnp.float32)],
        scratch_shapes=[pltpu.VMEM((2, cm), jnp.float32)],
        compiler_params=arb,
    )(x3, ss1, w1b, w9, gb2)

    z3t, ss3t = pl.pallas_call(
        functools.partial(_pw2_kernel, count=fm), grid=(nt,),
        in_specs=[pl.BlockSpec((bm, cm), lambda i: (i, 0)),
                  pl.BlockSpec((2, cm), lambda i: (0, 0)),
                  pl.BlockSpec((cm, co), lambda i: (0, 0)),
                  pl.BlockSpec((co, co), lambda i: (0, 0)),
                  pl.BlockSpec((2, co), lambda i: (0, 0))],
        out_specs=[pl.BlockSpec((co, bm), lambda i: (0, i)),
                   pl.BlockSpec((co, 2), lambda i: (0, 0))],
        out_shape=[jax.ShapeDtypeStruct((co, m), jnp.bfloat16),
                   jax.ShapeDtypeStruct((co, 2), jnp.float32)],
        scratch_shapes=[pltpu.VMEM((2, co), jnp.float32)],
        compiler_params=arb,
    )(z2.reshape(m, cm), ss2, w2b, eye, gb3)

    out3 = pl.pallas_call(
        _bn_res_kernel, grid=(n,),
        in_specs=[pl.BlockSpec((co, hw), lambda i: (0, i)),
                  pl.BlockSpec((co, 2), lambda i: (0, 0)),
                  pl.BlockSpec((None, c, hw), lambda i: (i, 0, 0))],
        out_specs=pl.BlockSpec((None, co, hw), lambda i: (i, 0, 0)),
        out_shape=jax.ShapeDtypeStruct((n, co, hw), jnp.float32),
        compiler_params=pltpu.CompilerParams(
            dimension_semantics=("parallel",)),
    )(z3t, ss3t, x3)

    return out3.reshape(n, co, h, w)
